# baseline (device time: 46213 ns/iter reference)
import jax
import jax.numpy as jnp
from jax import lax
from jax.experimental import pallas as pl
from jax.experimental.pallas import tpu as pltpu

N_DEV = 4
S = 2048
H = 8
DH = 128
D = H * DH
W = 128
QB = 256
KB = QB + 2 * W
NBLK = S // QB
SCALE = 0.08838834764831843
NEG = -1e9


def kernel(x, Wq, K_ext, V_ext, Wo):
    xs = x.reshape(S, D)
    k3 = K_ext.reshape(S, H, DH)
    v3 = V_ext.reshape(S, H, DH)

    def body(x_ref, wq_ref, k_ref, v_ref, wo_ref, out_ref,
             stg_ref, ekb_ref, evb_ref, qs_ref,
             local_sem, send_sems, recv_sems):
        my = lax.axis_index("i")
        left = lax.rem(my + N_DEV - 1, N_DEV)
        right = lax.rem(my + 1, N_DEV)

        def start_dmas(src3_ref):
            cs = [pltpu.make_async_copy(
                src3_ref.at[:, h, :],
                stg_ref.at[:, pl.ds(h * DH, DH)], local_sem)
                for h in range(H)]
            for c in cs:
                c.start()
            return cs

        k_copies = start_dmas(k_ref)

        barrier_sem = pltpu.get_barrier_semaphore()
        for nbr in (left, right):
            pl.semaphore_signal(barrier_sem, inc=1, device_id=(nbr,),
                                device_id_type=pl.DeviceIdType.MESH)
        pl.semaphore_wait(barrier_sem, 2)

        wq16 = wq_ref[...].astype(jnp.bfloat16)

        def q_proj(qb):
            qs_ref[pl.ds(qb * QB, QB), :] = (SCALE * jnp.dot(
                x_ref[pl.ds(qb * QB, QB), :].astype(jnp.bfloat16), wq16,
                preferred_element_type=jnp.float32)).astype(jnp.bfloat16)

        for qb in range(NBLK // 2):
            q_proj(qb)
        for c in k_copies:
            c.wait()
        ekb_ref[pl.ds(W, S), :] = stg_ref[...].astype(jnp.bfloat16)
        v_copies = start_dmas(v_ref)
        for qb in range(NBLK // 2, NBLK):
            q_proj(qb)
        for c in v_copies:
            c.wait()
        evb_ref[pl.ds(W, S), :] = stg_ref[...].astype(jnp.bfloat16)

        rdmas = [
            pltpu.make_async_remote_copy(
                src_ref=ekb_ref.at[pl.ds(S, W), :],
                dst_ref=ekb_ref.at[pl.ds(0, W), :],
                send_sem=send_sems.at[0], recv_sem=recv_sems.at[0],
                device_id=(right,), device_id_type=pl.DeviceIdType.MESH),
            pltpu.make_async_remote_copy(
                src_ref=evb_ref.at[pl.ds(S, W), :],
                dst_ref=evb_ref.at[pl.ds(0, W), :],
                send_sem=send_sems.at[1], recv_sem=recv_sems.at[1],
                device_id=(right,), device_id_type=pl.DeviceIdType.MESH),
            pltpu.make_async_remote_copy(
                src_ref=ekb_ref.at[pl.ds(W, W), :],
                dst_ref=ekb_ref.at[pl.ds(W + S, W), :],
                send_sem=send_sems.at[2], recv_sem=recv_sems.at[2],
                device_id=(left,), device_id_type=pl.DeviceIdType.MESH),
            pltpu.make_async_remote_copy(
                src_ref=evb_ref.at[pl.ds(W, W), :],
                dst_ref=evb_ref.at[pl.ds(W + S, W), :],
                send_sem=send_sems.at[3], recv_sem=recv_sems.at[3],
                device_id=(left,), device_id_type=pl.DeviceIdType.MESH),
        ]
        for r in rdmas:
            r.start()

        r_idx = lax.broadcasted_iota(jnp.int32, (QB, KB), 0)
        c_idx = lax.broadcasted_iota(jnp.int32, (QB, KB), 1)
        band = (c_idx >= r_idx) & (c_idx - r_idx <= 2 * W)
        bias_mid = jnp.where(band, 0.0, NEG).astype(jnp.float32)
        bias_lo = jnp.where((my == 0) & (c_idx < W), NEG, bias_mid)
        bias_hi = jnp.where((my == N_DEV - 1) & (c_idx >= KB - W),
                            NEG, bias_mid)

        wo16 = wo_ref[...].astype(jnp.bfloat16)

        def do_block(qb, bias):
            q_blk = qs_ref[pl.ds(qb * QB, QB), :]
            kb = ekb_ref[pl.ds(qb * QB, KB), :]
            vb = evb_ref[pl.ds(qb * QB, KB), :]
            ctx = []
            for h in range(H):
                qh = q_blk[:, h * DH:(h + 1) * DH]
                kh = kb[:, h * DH:(h + 1) * DH]
                s = lax.dot_general(
                    qh, kh, (((1,), (1,)), ((), ())),
                    preferred_element_type=jnp.float32)
                w = jnp.exp(s + bias)
                denom = jnp.sum(w, axis=-1, keepdims=True)
                ctx_h = jnp.dot(w.astype(jnp.bfloat16),
                                vb[:, h * DH:(h + 1) * DH],
                                preferred_element_type=jnp.float32)
                ctx.append((ctx_h * (1.0 / denom)).astype(jnp.bfloat16))
            ctx_blk = jnp.concatenate(ctx, axis=-1)
            out_ref[pl.ds(qb * QB, QB), :] = jnp.dot(
                ctx_blk, wo16, preferred_element_type=jnp.float32)

        for qb in range(1, NBLK - 1):
            do_block(qb, bias_mid)
        for r in rdmas:
            r.wait()
        do_block(0, bias_lo)
        do_block(NBLK - 1, bias_hi)

    out = pl.pallas_call(
        body,
        out_shape=jax.ShapeDtypeStruct((S, D), jnp.float32),
        in_specs=[
            pl.BlockSpec(memory_space=pltpu.VMEM),
            pl.BlockSpec(memory_space=pltpu.VMEM),
            pl.BlockSpec(memory_space=pl.ANY),
            pl.BlockSpec(memory_space=pl.ANY),
            pl.BlockSpec(memory_space=pltpu.VMEM),
        ],
        out_specs=pl.BlockSpec(memory_space=pltpu.VMEM),
        scratch_shapes=[
            pltpu.VMEM((S, D), jnp.float32),
            pltpu.VMEM((S + 2 * W, D), jnp.bfloat16),
            pltpu.VMEM((S + 2 * W, D), jnp.bfloat16),
            pltpu.VMEM((S, D), jnp.bfloat16),
            pltpu.SemaphoreType.DMA,
            pltpu.SemaphoreType.DMA((4,)),
            pltpu.SemaphoreType.DMA((4,)),
        ],
        compiler_params=pltpu.CompilerParams(
            collective_id=0, vmem_limit_bytes=100 * 1024 * 1024),
    )(xs, Wq, k3, v3, Wo)
    return out.reshape(1, S, D)


# device time: 40859 ns/iter; 1.1310x vs baseline; 1.1310x over previous
import jax
import jax.numpy as jnp
from jax import lax
from jax.experimental import pallas as pl
from jax.experimental.pallas import tpu as pltpu

N_DEV = 4
S = 2048
H = 8
DH = 128
D = H * DH
W = 128
QB = 256
KB = QB + 2 * W
NBLK = S // QB
SCALE = 0.08838834764831843
NEG = -1e9


def kernel(x, Wq, K_ext, V_ext, Wo):
    xs = x.reshape(S, D)
    k3 = K_ext.reshape(S, H, DH)
    v3 = V_ext.reshape(S, H, DH)

    def body(x_ref, wq_ref, k_ref, v_ref, wo_ref, out_ref,
             ek_ref, ev_ref, qs_ref, local_sem, send_sems, recv_sems):
        my = lax.axis_index("i")
        left = lax.rem(my + N_DEV - 1, N_DEV)
        right = lax.rem(my + 1, N_DEV)

        copies = []
        for h in range(H):
            copies.append(pltpu.make_async_copy(
                k_ref.at[:, h, :],
                ek_ref.at[pl.ds(W, S), pl.ds(h * DH, DH)], local_sem))
            copies.append(pltpu.make_async_copy(
                v_ref.at[:, h, :],
                ev_ref.at[pl.ds(W, S), pl.ds(h * DH, DH)], local_sem))
        for c in copies:
            c.start()

        barrier_sem = pltpu.get_barrier_semaphore()
        for nbr in (left, right):
            pl.semaphore_signal(barrier_sem, inc=1, device_id=(nbr,),
                                device_id_type=pl.DeviceIdType.MESH)
        pl.semaphore_wait(barrier_sem, 2)

        for qb in range(NBLK):
            qs_ref[pl.ds(qb * QB, QB), :] = SCALE * jnp.dot(
                x_ref[pl.ds(qb * QB, QB), :], wq_ref[...],
                preferred_element_type=jnp.float32)

        for c in copies:
            c.wait()

        rdmas = [
            pltpu.make_async_remote_copy(
                src_ref=ek_ref.at[pl.ds(S, W), :],
                dst_ref=ek_ref.at[pl.ds(0, W), :],
                send_sem=send_sems.at[0], recv_sem=recv_sems.at[0],
                device_id=(right,), device_id_type=pl.DeviceIdType.MESH),
            pltpu.make_async_remote_copy(
                src_ref=ev_ref.at[pl.ds(S, W), :],
                dst_ref=ev_ref.at[pl.ds(0, W), :],
                send_sem=send_sems.at[1], recv_sem=recv_sems.at[1],
                device_id=(right,), device_id_type=pl.DeviceIdType.MESH),
            pltpu.make_async_remote_copy(
                src_ref=ek_ref.at[pl.ds(W, W), :],
                dst_ref=ek_ref.at[pl.ds(W + S, W), :],
                send_sem=send_sems.at[2], recv_sem=recv_sems.at[2],
                device_id=(left,), device_id_type=pl.DeviceIdType.MESH),
            pltpu.make_async_remote_copy(
                src_ref=ev_ref.at[pl.ds(W, W), :],
                dst_ref=ev_ref.at[pl.ds(W + S, W), :],
                send_sem=send_sems.at[3], recv_sem=recv_sems.at[3],
                device_id=(left,), device_id_type=pl.DeviceIdType.MESH),
        ]
        for r in rdmas:
            r.start()

        r_idx = lax.broadcasted_iota(jnp.int32, (QB, KB), 0)
        c_idx = lax.broadcasted_iota(jnp.int32, (QB, KB), 1)
        band = (c_idx >= r_idx) & (c_idx - r_idx <= 2 * W)
        bias_mid = jnp.where(band, 0.0, NEG).astype(jnp.float32)
        bias_lo = jnp.where((my == 0) & (c_idx < W), NEG, bias_mid)
        bias_hi = jnp.where((my == N_DEV - 1) & (c_idx >= KB - W),
                            NEG, bias_mid)

        def do_block(qb, bias):
            q_blk = qs_ref[pl.ds(qb * QB, QB), :]
            kb = ek_ref[pl.ds(qb * QB, KB), :]
            vb = ev_ref[pl.ds(qb * QB, KB), :]
            ctx = []
            for h in range(H):
                qh = q_blk[:, h * DH:(h + 1) * DH]
                kh = kb[:, h * DH:(h + 1) * DH]
                s = lax.dot_general(
                    qh, kh, (((1,), (1,)), ((), ())),
                    preferred_element_type=jnp.float32)
                w = jnp.exp(s + bias)
                denom = jnp.sum(w, axis=-1, keepdims=True)
                ctx_h = jnp.dot(w, vb[:, h * DH:(h + 1) * DH],
                                preferred_element_type=jnp.float32)
                ctx.append(ctx_h * (1.0 / denom))
            ctx_blk = jnp.concatenate(ctx, axis=-1)
            out_ref[pl.ds(qb * QB, QB), :] = jnp.dot(
                ctx_blk, wo_ref[...], preferred_element_type=jnp.float32)

        for qb in range(1, NBLK - 1):
            do_block(qb, bias_mid)
        for r in rdmas:
            r.wait()
        do_block(0, bias_lo)
        do_block(NBLK - 1, bias_hi)

    out = pl.pallas_call(
        body,
        out_shape=jax.ShapeDtypeStruct((S, D), jnp.float32),
        in_specs=[
            pl.BlockSpec(memory_space=pltpu.VMEM),
            pl.BlockSpec(memory_space=pltpu.VMEM),
            pl.BlockSpec(memory_space=pl.ANY),
            pl.BlockSpec(memory_space=pl.ANY),
            pl.BlockSpec(memory_space=pltpu.VMEM),
        ],
        out_specs=pl.BlockSpec(memory_space=pltpu.VMEM),
        scratch_shapes=[
            pltpu.VMEM((S + 2 * W, D), jnp.float32),
            pltpu.VMEM((S + 2 * W, D), jnp.float32),
            pltpu.VMEM((S, D), jnp.float32),
            pltpu.SemaphoreType.DMA,
            pltpu.SemaphoreType.DMA((4,)),
            pltpu.SemaphoreType.DMA((4,)),
        ],
        compiler_params=pltpu.CompilerParams(
            collective_id=0, vmem_limit_bytes=100 * 1024 * 1024),
    )(xs, Wq, k3, v3, Wo)
    return out.reshape(1, S, D)
